# Initial kernel scaffold; baseline (speedup 1.0000x reference)
import jax
import jax.numpy as jnp
from jax.experimental import pallas as pl


def kernel(features, distsq, neighbour_indices):
    raise NotImplementedError("write your pallas kernel here")


# speedup vs baseline: 2.4652x; 2.4652x over previous
import functools

import jax
import jax.numpy as jnp
from jax import lax
from jax.experimental import pallas as pl
from jax.experimental.pallas import tpu as pltpu
from jax.experimental.pallas import tpu_sc as plsc

N_NODES = 10000
K_NEIGH = 32
D_FEAT = 128
SUBDIV = 3
SCALER = 10.0 * 1.0 * float(SUBDIV)

NUM_CORES = 2
NUM_SUBCORES = 16
NUM_WORKERS = NUM_CORES * NUM_SUBCORES

TOTAL_ROWS = N_NODES * K_NEIGH
ROWS_PER_WORKER = TOTAL_ROWS // NUM_WORKERS
CHUNK_ROWS = 200
CHUNK_STEPS = ROWS_PER_WORKER // CHUNK_ROWS
IDX_MINOR = 100

V_BLOCK = 400
TC_GRID = N_NODES // V_BLOCK


def _sc_gather(features, idx2d):
    mesh = plsc.VectorSubcoreMesh(core_axis_name="c", subcore_axis_name="s")

    @functools.partial(
        pl.kernel,
        out_type=jax.ShapeDtypeStruct((TOTAL_ROWS, D_FEAT), jnp.float32),
        mesh=mesh,
        scratch_types=[
            pltpu.VMEM((2, IDX_MINOR), jnp.int32),
            pltpu.VMEM((CHUNK_ROWS, D_FEAT), jnp.float32),
            pltpu.SemaphoreType.DMA,
        ],
    )
    def gather_kernel(feat_hbm, idx_hbm, out_hbm, idx_v, rows_v, sem):
        wid = lax.axis_index("s") * NUM_CORES + lax.axis_index("c")
        idx_row0 = wid * (ROWS_PER_WORKER // IDX_MINOR)
        out_row0 = wid * ROWS_PER_WORKER

        @pl.loop(0, CHUNK_STEPS)
        def _(it):
            pltpu.sync_copy(idx_hbm.at[pl.ds(idx_row0 + it * 2, 2)], idx_v)
            pltpu.async_copy(
                feat_hbm.at[idx_v.at[0]], rows_v.at[pl.ds(0, IDX_MINOR)], sem
            ).wait()
            pltpu.async_copy(
                feat_hbm.at[idx_v.at[1]], rows_v.at[pl.ds(IDX_MINOR, IDX_MINOR)], sem
            ).wait()
            pltpu.sync_copy(
                rows_v, out_hbm.at[pl.ds(out_row0 + it * CHUNK_ROWS, CHUNK_ROWS)]
            )

    return gather_kernel(features, idx2d)


def _tc_body(d_ref, g_ref, o_ref):
    g = g_ref[...].reshape(V_BLOCK, K_NEIGH, D_FEAT)
    dist = jnp.sqrt(d_ref[...] + 1e-6)
    outs = []
    for i in range(SUBDIV):
        offset = float(i) / float(SUBDIV)
        w = jnp.exp(-SCALER * (dist - offset) ** 2)
        wsum = jnp.sum(w, axis=1, keepdims=True) + 1e-6
        acc = jnp.sum(w[:, :, None] * g, axis=1)
        outs.append(acc / wsum)
    o_ref[...] = jnp.concatenate(outs, axis=-1)


def _tc_reduce(distsq, gathered):
    return pl.pallas_call(
        _tc_body,
        grid=(TC_GRID,),
        in_specs=[
            pl.BlockSpec((V_BLOCK, K_NEIGH), lambda b: (b, 0)),
            pl.BlockSpec((V_BLOCK * K_NEIGH, D_FEAT), lambda b: (b, 0)),
        ],
        out_specs=pl.BlockSpec((V_BLOCK, SUBDIV * D_FEAT), lambda b: (b, 0)),
        out_shape=jax.ShapeDtypeStruct((N_NODES, SUBDIV * D_FEAT), jnp.float32),
    )(distsq, gathered)


def kernel(features, distsq, neighbour_indices):
    idx2d = neighbour_indices.astype(jnp.int32).reshape(
        TOTAL_ROWS // IDX_MINOR, IDX_MINOR
    )
    gathered = _sc_gather(features, idx2d)
    return _tc_reduce(distsq, gathered)


# speedup vs baseline: 3.7783x; 1.5326x over previous
import functools
import math

import jax
import jax.numpy as jnp
from jax import lax
from jax.experimental import pallas as pl
from jax.experimental.pallas import tpu as pltpu
from jax.experimental.pallas import tpu_sc as plsc

N_NODES = 10000
K_NEIGH = 32
D_FEAT = 128
SUBDIV = 3
SCALER = 10.0 * 1.0 * float(SUBDIV)

NUM_CORES = 2
NUM_SUBCORES = 16
NUM_WORKERS = NUM_CORES * NUM_SUBCORES

TOTAL_ROWS = N_NODES * K_NEIGH
ROWS_PER_WORKER = TOTAL_ROWS // NUM_WORKERS
CHUNK_ROWS = 200
CHUNK_STEPS = ROWS_PER_WORKER // CHUNK_ROWS
IDX_MINOR = 100

V_BLOCK = 400
TC_GRID = N_NODES // V_BLOCK


def _sc_gather(features, idx2d):
    mesh = plsc.VectorSubcoreMesh(core_axis_name="c", subcore_axis_name="s")

    @functools.partial(
        pl.kernel,
        out_type=jax.ShapeDtypeStruct((TOTAL_ROWS, D_FEAT), jnp.float32),
        mesh=mesh,
        scratch_types=[
            pltpu.VMEM((2, 2, IDX_MINOR), jnp.int32),
            pltpu.VMEM((2, CHUNK_ROWS, D_FEAT), jnp.float32),
            pltpu.SemaphoreType.DMA,
            pltpu.SemaphoreType.DMA,
            pltpu.SemaphoreType.DMA,
            pltpu.SemaphoreType.DMA,
        ],
    )
    def gather_kernel(feat_hbm, idx_hbm, out_hbm, idx_v, rows_v, g0, g1, o0, o1):
        wid = lax.axis_index("s") * NUM_CORES + lax.axis_index("c")
        idx_row0 = wid * (ROWS_PER_WORKER // IDX_MINOR)
        out_row0 = wid * ROWS_PER_WORKER
        gsem = (g0, g1)
        osem = (o0, o1)

        def gathers(b, n):
            return [
                pltpu.make_async_copy(
                    feat_hbm.at[idx_v.at[b].at[j]],
                    rows_v.at[b].at[pl.ds(j * IDX_MINOR, IDX_MINOR)],
                    gsem[b],
                )
                for j in range(2)
            ]

        def start_chunk(b, n):
            pltpu.sync_copy(idx_hbm.at[pl.ds(idx_row0 + n * 2, 2)], idx_v.at[b])
            for c in gathers(b, n):
                c.start()

        def wait_gather(b, n):
            for c in gathers(b, n):
                c.wait()

        def out_copy(b, n):
            return pltpu.make_async_copy(
                rows_v.at[b],
                out_hbm.at[pl.ds(out_row0 + n * CHUNK_ROWS, CHUNK_ROWS)],
                osem[b],
            )

        start_chunk(0, 0)
        start_chunk(1, 1)

        @pl.loop(0, CHUNK_STEPS - 2, step=2)
        def _(n):
            wait_gather(0, n)
            out_copy(0, n).start()
            out_copy(0, n).wait()
            start_chunk(0, n + 2)
            wait_gather(1, n + 1)
            out_copy(1, n + 1).start()
            out_copy(1, n + 1).wait()
            start_chunk(1, n + 3)

        n_last = CHUNK_STEPS - 2
        wait_gather(0, n_last)
        out_copy(0, n_last).start()
        wait_gather(1, n_last + 1)
        out_copy(1, n_last + 1).start()
        out_copy(0, n_last).wait()
        out_copy(1, n_last + 1).wait()

    return gather_kernel(features, idx2d)


def _tc_body(d_ref, g_ref, o_ref):
    g = g_ref[...]
    dsq = d_ref[...].reshape(K_NEIGH, V_BLOCK) + 1e-6
    dist = jnp.sqrt(dsq)
    a = jnp.exp(-SCALER * dsq)
    t = jnp.exp((2.0 * SCALER / 3.0) * dist)
    c1 = float(math.exp(-SCALER / 9.0))
    c2 = float(math.exp(-SCALER * 4.0 / 9.0))
    ab = a[:, :, None]
    tb = t[:, :, None]
    p = ab * g
    q = p * tb
    r = q * tb
    w1 = a * t
    w2 = w1 * t
    rcp0 = 1.0 / (jnp.sum(a, axis=0) + 1e-6)
    rcp1 = c1 / (jnp.sum(w1, axis=0) * c1 + 1e-6)
    rcp2 = c2 / (jnp.sum(w2, axis=0) * c2 + 1e-6)
    out0 = jnp.sum(p, axis=0) * rcp0[:, None]
    out1 = jnp.sum(q, axis=0) * rcp1[:, None]
    out2 = jnp.sum(r, axis=0) * rcp2[:, None]
    o_ref[...] = jnp.concatenate([out0, out1, out2], axis=-1)


def _tc_reduce(distsq_t, gathered3):
    return pl.pallas_call(
        _tc_body,
        grid=(TC_GRID,),
        in_specs=[
            pl.BlockSpec((1, K_NEIGH, V_BLOCK), lambda b: (b, 0, 0)),
            pl.BlockSpec((K_NEIGH, V_BLOCK, D_FEAT), lambda b: (0, b, 0)),
        ],
        out_specs=pl.BlockSpec((V_BLOCK, SUBDIV * D_FEAT), lambda b: (b, 0)),
        out_shape=jax.ShapeDtypeStruct((N_NODES, SUBDIV * D_FEAT), jnp.float32),
    )(distsq_t, gathered3)


def kernel(features, distsq, neighbour_indices):
    idx2d = (
        neighbour_indices.astype(jnp.int32)
        .T.reshape(TOTAL_ROWS // IDX_MINOR, IDX_MINOR)
    )
    gathered = _sc_gather(features, idx2d)
    distsq_t = distsq.T.reshape(K_NEIGH, TC_GRID, V_BLOCK).transpose(1, 0, 2)
    return _tc_reduce(distsq_t, gathered.reshape(K_NEIGH, N_NODES, D_FEAT))


# speedup vs baseline: 4.0309x; 1.0668x over previous
import functools
import math

import jax
import jax.numpy as jnp
from jax import lax
from jax.experimental import pallas as pl
from jax.experimental.pallas import tpu as pltpu
from jax.experimental.pallas import tpu_sc as plsc

N_NODES = 10000
K_NEIGH = 32
D_FEAT = 128
SUBDIV = 3
SCALER = 10.0 * 1.0 * float(SUBDIV)

NUM_CORES = 2
NUM_SUBCORES = 16
NUM_WORKERS = NUM_CORES * NUM_SUBCORES

TOTAL_ROWS = N_NODES * K_NEIGH
CHUNK_ROWS = 200
IDX_MINOR = 100

N_CHUNKS = 5
V_CHUNK = N_NODES // N_CHUNKS
ROWS_PER_CHUNK = V_CHUNK * K_NEIGH
ROWS_PER_WORKER = ROWS_PER_CHUNK // NUM_WORKERS
CHUNK_STEPS = ROWS_PER_WORKER // CHUNK_ROWS

V_BLOCK = 400
TC_GRID = V_CHUNK // V_BLOCK


def _sc_gather(features, idx2d):
    mesh = plsc.VectorSubcoreMesh(core_axis_name="c", subcore_axis_name="s")

    @functools.partial(
        pl.kernel,
        out_type=jax.ShapeDtypeStruct((ROWS_PER_CHUNK, D_FEAT), jnp.float32),
        mesh=mesh,
        scratch_types=[
            pltpu.VMEM((2, 2, IDX_MINOR), jnp.int32),
            pltpu.VMEM((2, CHUNK_ROWS, D_FEAT), jnp.float32),
            pltpu.SemaphoreType.DMA,
            pltpu.SemaphoreType.DMA,
            pltpu.SemaphoreType.DMA,
            pltpu.SemaphoreType.DMA,
        ],
    )
    def gather_kernel(feat_hbm, idx_hbm, out_hbm, idx_v, rows_v, g0, g1, o0, o1):
        wid = lax.axis_index("s") * NUM_CORES + lax.axis_index("c")
        idx_row0 = wid * (ROWS_PER_WORKER // IDX_MINOR)
        out_row0 = wid * ROWS_PER_WORKER
        gsem = (g0, g1)
        osem = (o0, o1)

        def gathers(b, n):
            return [
                pltpu.make_async_copy(
                    feat_hbm.at[idx_v.at[b].at[j]],
                    rows_v.at[b].at[pl.ds(j * IDX_MINOR, IDX_MINOR)],
                    gsem[b],
                )
                for j in range(2)
            ]

        def start_chunk(b, n):
            pltpu.sync_copy(idx_hbm.at[pl.ds(idx_row0 + n * 2, 2)], idx_v.at[b])
            for c in gathers(b, n):
                c.start()

        def wait_gather(b, n):
            for c in gathers(b, n):
                c.wait()

        def out_copy(b, n):
            return pltpu.make_async_copy(
                rows_v.at[b],
                out_hbm.at[pl.ds(out_row0 + n * CHUNK_ROWS, CHUNK_ROWS)],
                osem[b],
            )

        start_chunk(0, 0)
        start_chunk(1, 1)

        @pl.loop(0, CHUNK_STEPS - 2, step=2)
        def _(n):
            wait_gather(0, n)
            out_copy(0, n).start()
            out_copy(0, n).wait()
            start_chunk(0, n + 2)
            wait_gather(1, n + 1)
            out_copy(1, n + 1).start()
            out_copy(1, n + 1).wait()
            start_chunk(1, n + 3)

        n_last = CHUNK_STEPS - 2
        wait_gather(0, n_last)
        out_copy(0, n_last).start()
        wait_gather(1, n_last + 1)
        out_copy(1, n_last + 1).start()
        out_copy(0, n_last).wait()
        out_copy(1, n_last + 1).wait()

    return gather_kernel(features, idx2d)


def _tc_body(d_ref, g_ref, o_ref):
    g = g_ref[...]
    dsq = d_ref[...].reshape(K_NEIGH, V_BLOCK) + 1e-6
    dist = jnp.sqrt(dsq)
    a = jnp.exp(-SCALER * dsq)
    t = jnp.exp((2.0 * SCALER / 3.0) * dist)
    c1 = float(math.exp(-SCALER / 9.0))
    c2 = float(math.exp(-SCALER * 4.0 / 9.0))
    ab = a[:, :, None]
    tb = t[:, :, None]
    p = ab * g
    q = p * tb
    r = q * tb
    w1 = a * t
    w2 = w1 * t
    rcp0 = 1.0 / (jnp.sum(a, axis=0) + 1e-6)
    rcp1 = c1 / (jnp.sum(w1, axis=0) * c1 + 1e-6)
    rcp2 = c2 / (jnp.sum(w2, axis=0) * c2 + 1e-6)
    out0 = jnp.sum(p, axis=0) * rcp0[:, None]
    out1 = jnp.sum(q, axis=0) * rcp1[:, None]
    out2 = jnp.sum(r, axis=0) * rcp2[:, None]
    o_ref[...] = jnp.concatenate([out0, out1, out2], axis=-1)


def _tc_reduce(distsq_t, gathered3):
    return pl.pallas_call(
        _tc_body,
        grid=(TC_GRID,),
        in_specs=[
            pl.BlockSpec((1, K_NEIGH, V_BLOCK), lambda b: (b, 0, 0)),
            pl.BlockSpec((K_NEIGH, V_BLOCK, D_FEAT), lambda b: (0, b, 0)),
        ],
        out_specs=pl.BlockSpec((V_BLOCK, SUBDIV * D_FEAT), lambda b: (b, 0)),
        out_shape=jax.ShapeDtypeStruct((V_CHUNK, SUBDIV * D_FEAT), jnp.float32),
    )(distsq_t, gathered3)


def kernel(features, distsq, neighbour_indices):
    idx_t = neighbour_indices.astype(jnp.int32).T
    dsq_t = distsq.T
    outs = []
    for ci in range(N_CHUNKS):
        sl = slice(ci * V_CHUNK, (ci + 1) * V_CHUNK)
        idx2d = idx_t[:, sl].reshape(ROWS_PER_CHUNK // IDX_MINOR, IDX_MINOR)
        gathered = _sc_gather(features, idx2d)
        distsq_t = dsq_t[:, sl].reshape(K_NEIGH, TC_GRID, V_BLOCK).transpose(1, 0, 2)
        outs.append(
            _tc_reduce(distsq_t, gathered.reshape(K_NEIGH, V_CHUNK, D_FEAT))
        )
    return jnp.concatenate(outs, axis=0)
